# baseline (speedup 1.0000x reference)
import jax
import jax.numpy as jnp
from jax import lax
from jax.experimental import pallas as pl
from jax.experimental.pallas import tpu as pltpu
from jax.experimental.pallas import tpu_sc as plsc

_N = 10000
_E = 320000
_D = 128
_NC = 2
_NS = 16
_K = 80
_RPT = 624
_RLAST = _N - (_NS - 1) * _RPT


def _init_rows(src_hbm, sh, s):
    @pl.when(s < _NS - 1)
    def _():
        pltpu.sync_copy(src_hbm.at[pl.ds(0, _RPT)], sh.at[pl.ds(s * _RPT, _RPT)])

    @pl.when(s == _NS - 1)
    def _():
        pltpu.sync_copy(src_hbm, sh.at[pl.ds((_NS - 1) * _RPT, _RLAST)])


def _writeout_rows(sh, out_hbm, c, s):
    @pl.when(s < _NS - 1)
    def _():
        pltpu.sync_copy(sh.at[pl.ds(s * _RPT, _RPT)],
                        out_hbm.at[c, pl.ds(s * _RPT, _RPT)])

    @pl.when(s == _NS - 1)
    def _():
        pltpu.sync_copy(sh.at[pl.ds((_NS - 1) * _RPT, _RLAST)],
                        out_hbm.at[c, pl.ds((_NS - 1) * _RPT, _RLAST)])


def _seg_mesh():
    return plsc.VectorSubcoreMesh(core_axis_name="c", subcore_axis_name="s",
                                  num_cores=_NC, num_subcores=_NS)


_NCH = _E // (_NC * _NS) // _K
_NP = _N + 16


def _hist_chunk(dsth_v, deg_t, lane):
    for l2 in range(_K // 16):
        d16 = dsth_v[pl.ds(l2 * 16, 16)]
        for l in range(16):
            idx = d16[l]
            bb = (idx >> 4) << 4
            lin = idx - bb
            vec = deg_t[pl.ds(bb, 16)]
            deg_t[pl.ds(bb, 16)] = vec + jnp.where(lane == lin, 1.0, 0.0)


def _make_segsum_body(with_deg):
    def body(z_hbm, src_hbm, dst_hbm, z0_hbm, *rest):
        if with_deg:
            (n0_hbm, out_hbm, deg_hbm, src0, src1, dst0, dst1, rows_v,
             deg_t, acc_sh, semS0, semS1, semD0, semD1, semG) = rest
        else:
            (out_hbm, src0, src1, dst0, dst1, rows_v,
             acc_sh, semS0, semS1, semD0, semD1, semG) = rest
        c = lax.axis_index("c")
        s = lax.axis_index("s")
        epw = _E // (_NC * _NS)
        _init_rows(z0_hbm, acc_sh, s)
        if with_deg:
            pltpu.sync_copy(n0_hbm, deg_t)
        plsc.subcore_barrier()
        base = (c * _NS + s) * epw
        srcs = (src0, src1)
        dsts = (dst0, dst1)
        semS = (semS0, semS1)
        semD = (semD0, semD1)

        def idx_load(j, b):
            off = base + j * _K
            pltpu.async_copy(src_hbm.at[pl.ds(off, _K)], srcs[b], semS[b])
            pltpu.async_copy(dst_hbm.at[pl.ds(off, _K)], dsts[b], semD[b])

        def idx_wait(j, b):
            off = base + j * _K
            pltpu.make_async_copy(src_hbm.at[pl.ds(off, _K)], srcs[b],
                                  semS[b]).wait()
            pltpu.make_async_copy(dst_hbm.at[pl.ds(off, _K)], dsts[b],
                                  semD[b]).wait()

        def chunk(j, b, prefetch):
            lane = lax.iota(jnp.int32, 16)
            idx_wait(j, b)
            pltpu.async_copy(z_hbm.at[srcs[b]], rows_v, semG).wait()
            pltpu.sync_copy(rows_v, acc_sh.at[dsts[b]], add=True)
            if with_deg:
                _hist_chunk(dsts[b], deg_t, lane)
            if prefetch:
                @pl.when(j + 2 < _NCH)
                def _():
                    idx_load(j + 2, b)

        idx_load(0, 0)
        idx_load(1, 1)

        def it(i2, carry):
            for b in range(2):
                chunk(i2 * 2 + b, b, True)
            return carry

        lax.fori_loop(0, _NCH // 2, it, 0)
        chunk(_NCH - 1, 0, False)
        plsc.subcore_barrier()
        _writeout_rows(acc_sh, out_hbm, c, s)
        if with_deg:
            pltpu.sync_copy(deg_t, deg_hbm.at[c, s])

    return body


def _segsum_deg(z, src, dst):
    f = pl.kernel(
        _make_segsum_body(True),
        out_type=[jax.ShapeDtypeStruct((_NC, _N, _D), jnp.float32),
                  jax.ShapeDtypeStruct((_NC, _NS, _NP), jnp.float32)],
        mesh=_seg_mesh(),
        scratch_types=[
            pltpu.VMEM((_K,), jnp.int32),
            pltpu.VMEM((_K,), jnp.int32),
            pltpu.VMEM((_K,), jnp.int32),
            pltpu.VMEM((_K,), jnp.int32),
            pltpu.VMEM((_K, _D), jnp.float32),
            pltpu.VMEM((_NP,), jnp.float32),
            pltpu.VMEM_SHARED((_NP, _D), jnp.float32),
            pltpu.SemaphoreType.DMA,
            pltpu.SemaphoreType.DMA,
            pltpu.SemaphoreType.DMA,
            pltpu.SemaphoreType.DMA,
            pltpu.SemaphoreType.DMA,
        ],
    )
    z0 = jnp.zeros((_RLAST, _D), jnp.float32)
    n0 = jnp.zeros((_NP,), jnp.float32)
    return f(z, src, dst, z0, n0)


def _segsum(z, src, dst):
    f = pl.kernel(
        _make_segsum_body(False),
        out_type=jax.ShapeDtypeStruct((_NC, _N, _D), jnp.float32),
        mesh=_seg_mesh(),
        scratch_types=[
            pltpu.VMEM((_K,), jnp.int32),
            pltpu.VMEM((_K,), jnp.int32),
            pltpu.VMEM((_K,), jnp.int32),
            pltpu.VMEM((_K,), jnp.int32),
            pltpu.VMEM((_K, _D), jnp.float32),
            pltpu.VMEM_SHARED((_NP, _D), jnp.float32),
            pltpu.SemaphoreType.DMA,
            pltpu.SemaphoreType.DMA,
            pltpu.SemaphoreType.DMA,
            pltpu.SemaphoreType.DMA,
            pltpu.SemaphoreType.DMA,
        ],
    )
    z0 = jnp.zeros((_RLAST, _D), jnp.float32)
    return f(z, src, dst, z0)


_BN = 1000


def _invd_body(degp_ref, o_ref):
    deg = jnp.sum(degp_ref[...], axis=(0, 1))[: _N]
    o_ref[...] = (1.0 / jnp.maximum(deg, 1.0)).reshape(_N, 1)


def _inv_degree(degp):
    return pl.pallas_call(
        _invd_body,
        grid=(1,),
        in_specs=[pl.BlockSpec((_NC, _NS, _NP), lambda i: (0, 0, 0))],
        out_specs=pl.BlockSpec((_N, 1), lambda i: (0, 0)),
        out_shape=jax.ShapeDtypeStruct((_N, 1), jnp.float32),
    )(degp)


def _mid_body(sp_ref, invd_ref, x_ref, b_ref, wl_ref, wr_ref, h_ref):
    sagg = sp_ref[0] + sp_ref[1]
    invd = invd_ref[...]
    mean = sagg * invd
    pre = (jnp.dot(mean, wl_ref[...], preferred_element_type=jnp.float32)
           + b_ref[...]
           + jnp.dot(x_ref[...], wr_ref[...], preferred_element_type=jnp.float32))
    h_ref[...] = jnp.maximum(pre, 0.0)


def _dense_mid(sp, invd, x, b1, wl, wr):
    n, din = x.shape
    h = wl.shape[1]
    return pl.pallas_call(
        _mid_body,
        grid=(n // _BN,),
        in_specs=[pl.BlockSpec((_NC, _BN, _D), lambda i: (0, i, 0)),
                  pl.BlockSpec((_BN, 1), lambda i: (i, 0)),
                  pl.BlockSpec((_BN, din), lambda i: (i, 0)),
                  pl.BlockSpec((1, h), lambda i: (0, 0)),
                  pl.BlockSpec((din, h), lambda i: (0, 0)),
                  pl.BlockSpec((din, h), lambda i: (0, 0))],
        out_specs=pl.BlockSpec((_BN, h), lambda i: (i, 0)),
        out_shape=jax.ShapeDtypeStruct((n, h), jnp.float32),
    )(sp, invd, x, b1.reshape(1, h), wl, wr)


def _out_body(sp_ref, invd_ref, h_ref, b_ref, wl_ref, wr_ref, o_ref):
    sagg = sp_ref[0] + sp_ref[1]
    invd = invd_ref[...]
    mean = sagg * invd
    o = (jnp.dot(mean, wl_ref[...], preferred_element_type=jnp.float32)
         + b_ref[...]
         + jnp.dot(h_ref[...], wr_ref[...], preferred_element_type=jnp.float32))
    m = jnp.max(o, axis=1, keepdims=True)
    lse = jnp.log(jnp.sum(jnp.exp(o - m), axis=1, keepdims=True)) + m
    o_ref[...] = o - lse


def _dense_out(sp, invd, h, b2, wl, wr):
    n, hd = h.shape
    dout = wl.shape[1]
    return pl.pallas_call(
        _out_body,
        grid=(n // _BN,),
        in_specs=[pl.BlockSpec((_NC, _BN, _D), lambda i: (0, i, 0)),
                  pl.BlockSpec((_BN, 1), lambda i: (i, 0)),
                  pl.BlockSpec((_BN, hd), lambda i: (i, 0)),
                  pl.BlockSpec((1, dout), lambda i: (0, 0)),
                  pl.BlockSpec((hd, dout), lambda i: (0, 0)),
                  pl.BlockSpec((hd, dout), lambda i: (0, 0))],
        out_specs=pl.BlockSpec((_BN, dout), lambda i: (i, 0)),
        out_shape=jax.ShapeDtypeStruct((n, dout), jnp.float32),
    )(sp, invd, h, b2.reshape(1, dout), wl, wr)


def kernel(x, edge_index, W1_l, b1, W1_r, W2_l, b2, W2_r):
    src = edge_index[0]
    dst = edge_index[1]
    s1p, degp = _segsum_deg(x, src, dst)
    invd = _inv_degree(degp)
    h = _dense_mid(s1p, invd, x, b1, W1_l, W1_r)
    s2p = _segsum(h, src, dst)
    return _dense_out(s2p, invd, h, b2, W2_l, W2_r)


# speedup vs baseline: 1.3199x; 1.3199x over previous
import jax
import jax.numpy as jnp
from jax import lax
from jax.experimental import pallas as pl
from jax.experimental.pallas import tpu as pltpu
from jax.experimental.pallas import tpu_sc as plsc

_N = 10000
_E = 320000
_D = 128
_NC = 2
_NS = 16
_K = 80
_RPT = 624
_RLAST = _N - (_NS - 1) * _RPT


def _init_rows(src_hbm, sh, s):
    @pl.when(s < _NS - 1)
    def _():
        pltpu.sync_copy(src_hbm.at[pl.ds(0, _RPT)], sh.at[pl.ds(s * _RPT, _RPT)])

    @pl.when(s == _NS - 1)
    def _():
        pltpu.sync_copy(src_hbm, sh.at[pl.ds((_NS - 1) * _RPT, _RLAST)])


def _writeout_rows(sh, out_hbm, c, s):
    @pl.when(s < _NS - 1)
    def _():
        pltpu.sync_copy(sh.at[pl.ds(s * _RPT, _RPT)],
                        out_hbm.at[c, pl.ds(s * _RPT, _RPT)])

    @pl.when(s == _NS - 1)
    def _():
        pltpu.sync_copy(sh.at[pl.ds((_NS - 1) * _RPT, _RLAST)],
                        out_hbm.at[c, pl.ds((_NS - 1) * _RPT, _RLAST)])


def _seg_mesh():
    return plsc.VectorSubcoreMesh(core_axis_name="c", subcore_axis_name="s",
                                  num_cores=_NC, num_subcores=_NS)


_NCH = _E // (_NC * _NS) // _K
_NP = _N + 16


def _hist_chunk(dsth_v, deg_t, lane):
    for l2 in range(_K // 16):
        d16 = dsth_v[pl.ds(l2 * 16, 16)]
        for l in range(16):
            idx = d16[l]
            bb = (idx >> 4) << 4
            lin = idx - bb
            vec = deg_t[pl.ds(bb, 16)]
            deg_t[pl.ds(bb, 16)] = vec + jnp.where(lane == lin, 1.0, 0.0)


def _make_segsum_body(with_deg):
    def body(z_hbm, src_hbm, dst_hbm, z0_hbm, *rest):
        if with_deg:
            (n0_hbm, out_hbm, deg_hbm, src0, src1, dst0, dst1, rows0, rows1,
             deg_t, acc_sh, semS0, semS1, semD0, semD1, semG0, semG1) = rest
        else:
            (out_hbm, src0, src1, dst0, dst1, rows0, rows1,
             acc_sh, semS0, semS1, semD0, semD1, semG0, semG1) = rest
        c = lax.axis_index("c")
        s = lax.axis_index("s")
        epw = _E // (_NC * _NS)
        _init_rows(z0_hbm, acc_sh, s)
        if with_deg:
            pltpu.sync_copy(n0_hbm, deg_t)
        plsc.subcore_barrier()
        base = (c * _NS + s) * epw
        srcs = (src0, src1)
        dsts = (dst0, dst1)
        rows = (rows0, rows1)
        semS = (semS0, semS1)
        semD = (semD0, semD1)
        semG = (semG0, semG1)

        def idx_load(j, b):
            off = base + j * _K
            pltpu.async_copy(src_hbm.at[pl.ds(off, _K)], srcs[b], semS[b])
            pltpu.async_copy(dst_hbm.at[pl.ds(off, _K)], dsts[b], semD[b])

        def idx_wait(j, b):
            off = base + j * _K
            pltpu.make_async_copy(src_hbm.at[pl.ds(off, _K)], srcs[b],
                                  semS[b]).wait()
            pltpu.make_async_copy(dst_hbm.at[pl.ds(off, _K)], dsts[b],
                                  semD[b]).wait()

        def chunk(j, b):
            lane = lax.iota(jnp.int32, 16)

            @pl.when(j + 1 < _NCH)
            def _():
                idx_wait(j + 1, 1 - b)
                pltpu.async_copy(z_hbm.at[srcs[1 - b]], rows[1 - b],
                                 semG[1 - b])

            pltpu.make_async_copy(z_hbm.at[srcs[b]], rows[b], semG[b]).wait()
            pltpu.sync_copy(rows[b], acc_sh.at[dsts[b]], add=True)
            if with_deg:
                _hist_chunk(dsts[b], deg_t, lane)

            @pl.when(j + 2 < _NCH)
            def _():
                idx_load(j + 2, b)

        idx_load(0, 0)
        idx_load(1, 1)
        idx_wait(0, 0)
        pltpu.async_copy(z_hbm.at[src0], rows0, semG0)

        def it(i2, carry):
            for b in range(2):
                chunk(i2 * 2 + b, b)
            return carry

        lax.fori_loop(0, _NCH // 2, it, 0)
        chunk(_NCH - 1, 0)
        plsc.subcore_barrier()
        _writeout_rows(acc_sh, out_hbm, c, s)
        if with_deg:
            pltpu.sync_copy(deg_t, deg_hbm.at[c, s])

    return body


def _segsum_deg(z, src, dst):
    f = pl.kernel(
        _make_segsum_body(True),
        out_type=[jax.ShapeDtypeStruct((_NC, _N, _D), jnp.float32),
                  jax.ShapeDtypeStruct((_NC, _NS, _NP), jnp.float32)],
        mesh=_seg_mesh(),
        scratch_types=[
            pltpu.VMEM((_K,), jnp.int32),
            pltpu.VMEM((_K,), jnp.int32),
            pltpu.VMEM((_K,), jnp.int32),
            pltpu.VMEM((_K,), jnp.int32),
            pltpu.VMEM((_K, _D), jnp.float32),
            pltpu.VMEM((_K, _D), jnp.float32),
            pltpu.VMEM((_NP,), jnp.float32),
            pltpu.VMEM_SHARED((_NP, _D), jnp.float32),
            pltpu.SemaphoreType.DMA,
            pltpu.SemaphoreType.DMA,
            pltpu.SemaphoreType.DMA,
            pltpu.SemaphoreType.DMA,
            pltpu.SemaphoreType.DMA,
            pltpu.SemaphoreType.DMA,
        ],
    )
    z0 = jnp.zeros((_RLAST, _D), jnp.float32)
    n0 = jnp.zeros((_NP,), jnp.float32)
    return f(z, src, dst, z0, n0)


def _segsum(z, src, dst):
    f = pl.kernel(
        _make_segsum_body(False),
        out_type=jax.ShapeDtypeStruct((_NC, _N, _D), jnp.float32),
        mesh=_seg_mesh(),
        scratch_types=[
            pltpu.VMEM((_K,), jnp.int32),
            pltpu.VMEM((_K,), jnp.int32),
            pltpu.VMEM((_K,), jnp.int32),
            pltpu.VMEM((_K,), jnp.int32),
            pltpu.VMEM((_K, _D), jnp.float32),
            pltpu.VMEM((_K, _D), jnp.float32),
            pltpu.VMEM_SHARED((_NP, _D), jnp.float32),
            pltpu.SemaphoreType.DMA,
            pltpu.SemaphoreType.DMA,
            pltpu.SemaphoreType.DMA,
            pltpu.SemaphoreType.DMA,
            pltpu.SemaphoreType.DMA,
            pltpu.SemaphoreType.DMA,
        ],
    )
    z0 = jnp.zeros((_RLAST, _D), jnp.float32)
    return f(z, src, dst, z0)


_BN = 1000


def _invd_body(degp_ref, o_ref):
    deg = jnp.sum(degp_ref[...], axis=(0, 1))[: _N]
    o_ref[...] = (1.0 / jnp.maximum(deg, 1.0)).reshape(_N, 1)


def _inv_degree(degp):
    return pl.pallas_call(
        _invd_body,
        grid=(1,),
        in_specs=[pl.BlockSpec((_NC, _NS, _NP), lambda i: (0, 0, 0))],
        out_specs=pl.BlockSpec((_N, 1), lambda i: (0, 0)),
        out_shape=jax.ShapeDtypeStruct((_N, 1), jnp.float32),
    )(degp)


def _mid_body(sp_ref, invd_ref, x_ref, b_ref, wl_ref, wr_ref, h_ref):
    sagg = sp_ref[0] + sp_ref[1]
    invd = invd_ref[...]
    mean = sagg * invd
    pre = (jnp.dot(mean, wl_ref[...], preferred_element_type=jnp.float32)
           + b_ref[...]
           + jnp.dot(x_ref[...], wr_ref[...], preferred_element_type=jnp.float32))
    h_ref[...] = jnp.maximum(pre, 0.0)


def _dense_mid(sp, invd, x, b1, wl, wr):
    n, din = x.shape
    h = wl.shape[1]
    return pl.pallas_call(
        _mid_body,
        grid=(n // _BN,),
        in_specs=[pl.BlockSpec((_NC, _BN, _D), lambda i: (0, i, 0)),
                  pl.BlockSpec((_BN, 1), lambda i: (i, 0)),
                  pl.BlockSpec((_BN, din), lambda i: (i, 0)),
                  pl.BlockSpec((1, h), lambda i: (0, 0)),
                  pl.BlockSpec((din, h), lambda i: (0, 0)),
                  pl.BlockSpec((din, h), lambda i: (0, 0))],
        out_specs=pl.BlockSpec((_BN, h), lambda i: (i, 0)),
        out_shape=jax.ShapeDtypeStruct((n, h), jnp.float32),
    )(sp, invd, x, b1.reshape(1, h), wl, wr)


def _out_body(sp_ref, invd_ref, h_ref, b_ref, wl_ref, wr_ref, o_ref):
    sagg = sp_ref[0] + sp_ref[1]
    invd = invd_ref[...]
    mean = sagg * invd
    o = (jnp.dot(mean, wl_ref[...], preferred_element_type=jnp.float32)
         + b_ref[...]
         + jnp.dot(h_ref[...], wr_ref[...], preferred_element_type=jnp.float32))
    m = jnp.max(o, axis=1, keepdims=True)
    lse = jnp.log(jnp.sum(jnp.exp(o - m), axis=1, keepdims=True)) + m
    o_ref[...] = o - lse


def _dense_out(sp, invd, h, b2, wl, wr):
    n, hd = h.shape
    dout = wl.shape[1]
    return pl.pallas_call(
        _out_body,
        grid=(n // _BN,),
        in_specs=[pl.BlockSpec((_NC, _BN, _D), lambda i: (0, i, 0)),
                  pl.BlockSpec((_BN, 1), lambda i: (i, 0)),
                  pl.BlockSpec((_BN, hd), lambda i: (i, 0)),
                  pl.BlockSpec((1, dout), lambda i: (0, 0)),
                  pl.BlockSpec((hd, dout), lambda i: (0, 0)),
                  pl.BlockSpec((hd, dout), lambda i: (0, 0))],
        out_specs=pl.BlockSpec((_BN, dout), lambda i: (i, 0)),
        out_shape=jax.ShapeDtypeStruct((n, dout), jnp.float32),
    )(sp, invd, h, b2.reshape(1, dout), wl, wr)


def kernel(x, edge_index, W1_l, b1, W1_r, W2_l, b2, W2_r):
    src = edge_index[0]
    dst = edge_index[1]
    s1p, degp = _segsum_deg(x, src, dst)
    invd = _inv_degree(degp)
    h = _dense_mid(s1p, invd, x, b1, W1_l, W1_r)
    s2p = _segsum(h, src, dst)
    return _dense_out(s2p, invd, h, b2, W2_l, W2_r)
